# baseline (speedup 1.0000x reference)
import jax
import jax.numpy as jnp
import numpy as np
from jax import lax
from jax.experimental import pallas as pl
from jax.experimental.pallas import tpu as pltpu
from jax.experimental.pallas import tpu_sc as plsc

B = 4
N = 16800
L = 16
NC = 1
NS = 16
WPB = 4
C = N // WPB
U = 2
NV = -(-C // L)
VAR0 = np.float32(0.1)
NEG_INF = np.float32(-np.inf)
IMAX = np.int32(2**31 - 1)

_MESH = plsc.VectorSubcoreMesh(
    core_axis_name="c", subcore_axis_name="s", num_cores=NC, num_subcores=NS
)


def _sc_body(scores_hbm, out_hbm, sbuf, mstage, istage, mload, iload, tmpf, tmpi):
    s = lax.axis_index("s")
    g = s // WPB
    w = s % WPB
    b = g
    base = w * C

    start = (2 * b + 1) * N + base
    a0 = (start // 8) * 8
    rem = start - a0
    pltpu.sync_copy(scores_hbm.at[pl.ds(a0, C + 4)], sbuf.at[pl.ds(0, C + 4)])

    lane = lax.iota(jnp.int32, L)
    lane_r = lane + rem

    def scan_vec(j, carry):
        run_max, run_idx = carry
        n = j * L + lane
        v = plsc.load_gather(sbuf, [j * L + lane_r])
        v = jnp.where(v > 0.0, v, NEG_INF)
        upd = v > run_max
        return jnp.where(upd, v, run_max), jnp.where(upd, base + n, run_idx)

    def step(i, chains):
        return tuple(scan_vec(i * U + k, chains[k]) for k in range(U))

    init = tuple(
        (jnp.full((L,), NEG_INF, jnp.float32), jnp.zeros((L,), jnp.int32))
        for _ in range(U)
    )
    nfull = (NV - 1) // U
    chains = lax.fori_loop(0, nfull, step, init)

    chains = tuple(
        scan_vec(nfull * U + k, chains[k]) if nfull * U + k < NV - 1 else chains[k]
        for k in range(U)
    )

    run_max, run_idx = chains[0]
    for m2, i2 in chains[1:]:
        upd = (m2 > run_max) | ((m2 == run_max) & (i2 < run_idx))
        run_max = jnp.where(upd, m2, run_max)
        run_idx = jnp.where(upd, i2, run_idx)

    n = (NV - 1) * L + lane
    v = plsc.load_gather(sbuf, [jnp.minimum(n, C - 1) + rem])
    v = jnp.where((v > 0.0) & (n < C), v, NEG_INF)
    upd = (v > run_max) | ((v == run_max) & (base + n < run_idx))
    run_max = jnp.where(upd, v, run_max)
    run_idx = jnp.where(upd, base + n, run_idx)

    tmpf[...] = run_max
    tmpi[...] = run_idx
    pltpu.sync_copy(tmpf, mstage.at[pl.ds(s * L, L)])
    pltpu.sync_copy(tmpi, istage.at[pl.ds(s * L, L)])
    plsc.subcore_barrier()

    @pl.when(w == 0)
    def _():
        pltpu.sync_copy(mstage.at[pl.ds(g * WPB * L, WPB * L)], mload)
        pltpu.sync_copy(istage.at[pl.ds(g * WPB * L, WPB * L)], iload)
        best_m = mload[pl.ds(0, L)]
        best_i = iload[pl.ds(0, L)]
        for k in range(1, WPB):
            m2 = mload[pl.ds(k * L, L)]
            i2 = iload[pl.ds(k * L, L)]
            upd = (m2 > best_m) | ((m2 == best_m) & (i2 < best_i))
            best_m = jnp.where(upd, m2, best_m)
            best_i = jnp.where(upd, i2, best_i)
        top = jnp.max(best_m, axis=0)
        cand = jnp.where(best_m == top, best_i, IMAX)
        tmpi[...] = jnp.min(cand, keepdims=True) + jnp.zeros((L,), jnp.int32)
        pltpu.sync_copy(tmpi, out_hbm.at[pl.ds(b * L, L)])


_sc_call = pl.kernel(
    _sc_body,
    out_type=jax.ShapeDtypeStruct((B * L,), jnp.int32),
    mesh=_MESH,
    compiler_params=pltpu.CompilerParams(
        needs_layout_passes=False, use_tc_tiling_on_sc=False
    ),
    scratch_types=[
        pltpu.VMEM((NV * L + 8,), jnp.float32),
        pltpu.VMEM_SHARED((NS * L,), jnp.float32),
        pltpu.VMEM_SHARED((NS * L,), jnp.int32),
        pltpu.VMEM((WPB * L,), jnp.float32),
        pltpu.VMEM((WPB * L,), jnp.int32),
        pltpu.VMEM((L,), jnp.float32),
        pltpu.VMEM((L,), jnp.int32),
    ],
)


_BLK = 128


def _tc_body(idx_ref, *refs):
    landms_refs = refs[:B]
    priors_refs = refs[B : 2 * B]
    out_ref = refs[2 * B]
    contract = (((1,), (1,)), ((), ()))
    kpar = lax.broadcasted_iota(jnp.int32, (1, L), 1) & 1
    nio = lax.broadcasted_iota(jnp.int32, (1, _BLK), 1)
    rows = []
    for b in range(B):
        rel = lax.rem(idx_ref[b * L], _BLK)
        mask1 = (nio == rel).astype(jnp.float32)
        lv = lax.dot_general(
            mask1, landms_refs[b][:, b, :], contract,
            precision=lax.Precision.HIGHEST,
            preferred_element_type=jnp.float32,
        )
        pr = lax.dot_general(
            mask1, priors_refs[b][...], contract,
            precision=lax.Precision.HIGHEST,
            preferred_element_type=jnp.float32,
        )
        lv16 = jnp.concatenate([lv, jnp.zeros((1, L - 10), jnp.float32)], axis=1)
        pxy = jnp.where(kpar == 0, pr[:, 0:1], pr[:, 1:2])
        pwh = jnp.where(kpar == 0, pr[:, 2:3], pr[:, 3:4])
        rows.append(pxy + lv16 * VAR0 * pwh)
    out_ref[...] = jnp.concatenate(rows, axis=0)


def _lm_spec(b):
    return pl.BlockSpec(
        (10, B, _BLK), lambda i, idx_ref: (0, 0, idx_ref[b * L] // _BLK)
    )


def _pr_spec(b):
    return pl.BlockSpec((B, _BLK), lambda i, idx_ref: (0, idx_ref[b * L] // _BLK))


_tc_call = pl.pallas_call(
    _tc_body,
    grid_spec=pltpu.PrefetchScalarGridSpec(
        num_scalar_prefetch=1,
        grid=(1,),
        in_specs=[_lm_spec(b) for b in range(B)] + [_pr_spec(b) for b in range(B)],
        out_specs=pl.BlockSpec((B, L), lambda i, idx_ref: (0, 0)),
    ),
    out_shape=jax.ShapeDtypeStruct((B, L), jnp.float32),
)


def kernel(loc, conf, landms, priors):
    del loc
    conf_f = conf.transpose(0, 2, 1).reshape(-1)
    idx_arr = _sc_call(conf_f)
    landms_t = landms.transpose(2, 0, 1)
    priors_t = priors.transpose(1, 0)
    out = _tc_call(idx_arr, *([landms_t] * B), *([priors_t] * B))
    return out[:, :10]


# speedup vs baseline: 1.0017x; 1.0017x over previous
import jax
import jax.numpy as jnp
import numpy as np
from jax import lax
from jax.experimental import pallas as pl
from jax.experimental.pallas import tpu as pltpu
from jax.experimental.pallas import tpu_sc as plsc

B = 4
N = 16800
L = 16
NC = 1
NS = 16
WPB = 4
C = N // WPB
U = 2
NV = -(-C // L)
VAR0 = np.float32(0.1)
NEG_INF = np.float32(-np.inf)
IMAX = np.int32(2**31 - 1)

_MESH = plsc.VectorSubcoreMesh(
    core_axis_name="c", subcore_axis_name="s", num_cores=NC, num_subcores=NS
)


def _sc_body(scores_hbm, out_hbm, sbuf, mstage, istage, mload, iload, tmpf, tmpi):
    s = lax.axis_index("s")
    g = s // WPB
    w = s % WPB
    b = g
    base = w * C

    start = (2 * b + 1) * N + base
    a0 = (start // 8) * 8
    rem = start - a0
    pltpu.sync_copy(scores_hbm.at[pl.ds(a0, C + 4)], sbuf.at[pl.ds(0, C + 4)])

    lane = lax.iota(jnp.int32, L)
    lane_r = lane + rem

    def scan_vec(j, carry):
        run_max, run_idx = carry
        n = j * L + lane
        v = plsc.load_gather(sbuf, [j * L + lane_r])
        v = jnp.where(v > 0.0, v, NEG_INF)
        upd = v > run_max
        return jnp.where(upd, v, run_max), jnp.where(upd, base + n, run_idx)

    def step(i, chains):
        return tuple(scan_vec(i * U + k, chains[k]) for k in range(U))

    init = tuple(
        (jnp.full((L,), NEG_INF, jnp.float32), jnp.zeros((L,), jnp.int32))
        for _ in range(U)
    )
    nfull = (NV - 1) // U
    chains = lax.fori_loop(0, nfull, step, init)

    chains = tuple(
        scan_vec(nfull * U + k, chains[k]) if nfull * U + k < NV - 1 else chains[k]
        for k in range(U)
    )

    run_max, run_idx = chains[0]
    for m2, i2 in chains[1:]:
        upd = (m2 > run_max) | ((m2 == run_max) & (i2 < run_idx))
        run_max = jnp.where(upd, m2, run_max)
        run_idx = jnp.where(upd, i2, run_idx)

    n = (NV - 1) * L + lane
    v = plsc.load_gather(sbuf, [jnp.minimum(n, C - 1) + rem])
    v = jnp.where((v > 0.0) & (n < C), v, NEG_INF)
    upd = (v > run_max) | ((v == run_max) & (base + n < run_idx))
    run_max = jnp.where(upd, v, run_max)
    run_idx = jnp.where(upd, base + n, run_idx)

    tmpf[...] = run_max
    tmpi[...] = run_idx
    pltpu.sync_copy(tmpf, mstage.at[pl.ds(s * L, L)])
    pltpu.sync_copy(tmpi, istage.at[pl.ds(s * L, L)])
    plsc.subcore_barrier()

    @pl.when(w == 0)
    def _():
        pltpu.sync_copy(mstage.at[pl.ds(g * WPB * L, WPB * L)], mload)
        pltpu.sync_copy(istage.at[pl.ds(g * WPB * L, WPB * L)], iload)
        best_m = mload[pl.ds(0, L)]
        best_i = iload[pl.ds(0, L)]
        for k in range(1, WPB):
            m2 = mload[pl.ds(k * L, L)]
            i2 = iload[pl.ds(k * L, L)]
            upd = (m2 > best_m) | ((m2 == best_m) & (i2 < best_i))
            best_m = jnp.where(upd, m2, best_m)
            best_i = jnp.where(upd, i2, best_i)
        top = jnp.max(best_m, axis=0)
        cand = jnp.where(best_m == top, best_i, IMAX)
        tmpi[...] = jnp.min(cand, keepdims=True) + jnp.zeros((L,), jnp.int32)
        pltpu.sync_copy(tmpi, out_hbm.at[pl.ds(b * L, L)])


_sc_call = pl.kernel(
    _sc_body,
    out_type=jax.ShapeDtypeStruct((B * L,), jnp.int32),
    mesh=_MESH,
    compiler_params=pltpu.CompilerParams(
        needs_layout_passes=False, use_tc_tiling_on_sc=False
    ),
    scratch_types=[
        pltpu.VMEM((NV * L + 8,), jnp.float32),
        pltpu.VMEM_SHARED((NS * L,), jnp.float32),
        pltpu.VMEM_SHARED((NS * L,), jnp.int32),
        pltpu.VMEM((WPB * L,), jnp.float32),
        pltpu.VMEM((WPB * L,), jnp.int32),
        pltpu.VMEM((L,), jnp.float32),
        pltpu.VMEM((L,), jnp.int32),
    ],
)


_BLK = 128


def _tc_body(idx_ref, *refs):
    landms_refs = refs[:B]
    priors_refs = refs[B : 2 * B]
    out_ref = refs[2 * B]
    contract = (((1,), (1,)), ((), ()))
    kpar = lax.broadcasted_iota(jnp.int32, (1, L), 1) & 1
    nio = lax.broadcasted_iota(jnp.int32, (1, _BLK), 1)
    rows = []
    for b in range(B):
        idx = idx_ref[b * L]
        rel = lax.rem(idx, _BLK)
        mask1 = (nio == rel).astype(jnp.float32)
        valid = (idx - rel + nio) < N
        lmat = jnp.where(valid, landms_refs[b][:, b, :], 0.0)
        pmat = jnp.where(valid, priors_refs[b][...], 0.0)
        lv = lax.dot_general(
            mask1, lmat, contract,
            precision=lax.Precision.HIGHEST,
            preferred_element_type=jnp.float32,
        )
        pr = lax.dot_general(
            mask1, pmat, contract,
            precision=lax.Precision.HIGHEST,
            preferred_element_type=jnp.float32,
        )
        lv16 = jnp.concatenate([lv, jnp.zeros((1, L - 10), jnp.float32)], axis=1)
        pxy = jnp.where(kpar == 0, pr[:, 0:1], pr[:, 1:2])
        pwh = jnp.where(kpar == 0, pr[:, 2:3], pr[:, 3:4])
        rows.append(pxy + lv16 * VAR0 * pwh)
    out_ref[...] = jnp.concatenate(rows, axis=0)


def _lm_spec(b):
    return pl.BlockSpec(
        (10, B, _BLK), lambda i, idx_ref: (0, 0, idx_ref[b * L] // _BLK)
    )


def _pr_spec(b):
    return pl.BlockSpec((B, _BLK), lambda i, idx_ref: (0, idx_ref[b * L] // _BLK))


_tc_call = pl.pallas_call(
    _tc_body,
    grid_spec=pltpu.PrefetchScalarGridSpec(
        num_scalar_prefetch=1,
        grid=(1,),
        in_specs=[_lm_spec(b) for b in range(B)] + [_pr_spec(b) for b in range(B)],
        out_specs=pl.BlockSpec((B, L), lambda i, idx_ref: (0, 0)),
    ),
    out_shape=jax.ShapeDtypeStruct((B, L), jnp.float32),
)


def kernel(loc, conf, landms, priors):
    del loc
    conf_f = conf.transpose(0, 2, 1).reshape(-1)
    idx_arr = _sc_call(conf_f)
    landms_t = landms.transpose(2, 0, 1)
    priors_t = priors.transpose(1, 0)
    out = _tc_call(idx_arr, *([landms_t] * B), *([priors_t] * B))
    return out[:, :10]
